# baseline (speedup 1.0000x reference)
import jax
import jax.numpy as jnp
from jax.experimental import pallas as pl
from jax.experimental.pallas import tpu as pltpu


def _nms_kernel(masks_ref, bb_ref, masks_hbm_ref, out_ref, acc_ref, s2_ref, sem, *, n, k_steps, hb, img_h, img_w):
    k = pl.program_id(0)

    @pl.when(k == 0)
    def _init():
        acc_ref[...] = jnp.zeros_like(acc_ref)

    @pl.when(k == 4)
    def _s2():
        b = bb_ref[...]
        w = b[:, 2:3] - b[:, 0:1]
        h = b[:, 3:4] - b[:, 1:2]
        keep1 = (h * (1.0 / img_h)) <= 0.95
        keep2 = (w * (1.0 / img_w)) <= 0.95
        ii = jax.lax.broadcasted_iota(jnp.int32, (n, n), 0)
        jj = jax.lax.broadcasted_iota(jnp.int32, (n, n), 1)
        tri = jnp.where(jj <= ii, 1.0, 0.0).astype(jnp.bfloat16)
        keep1f = keep1.astype(jnp.bfloat16)
        rank1 = (
            jax.lax.dot_general(
                tri, keep1f, (((1,), (0,)), ((), ())),
                preferred_element_type=jnp.float32,
            )
            - 1.0
        )
        oh = jnp.where(jj.astype(jnp.float32) == rank1, 1.0, 0.0).astype(
            jnp.bfloat16
        )
        keep2f = keep2.astype(jnp.bfloat16)
        gathered = jax.lax.dot_general(
            oh, keep2f, (((1,), (0,)), ((), ())),
            preferred_element_type=jnp.float32,
        )
        s2_ref[...] = jnp.where(keep1, gathered, 0.0)

    blk = masks_ref[...]
    a = blk.reshape(blk.shape[0], blk.shape[1] * blk.shape[2]).astype(jnp.bfloat16)
    part = jax.lax.dot_general(
        a, a, (((1,), (1,)), ((), ())), preferred_element_type=jnp.float32
    )
    acc_ref[...] += part

    @pl.when(k == k_steps - 1)
    def _epilogue():
        inter = acc_ref[...]
        s2_col = s2_ref[...]
        ii = jax.lax.broadcasted_iota(jnp.int32, (n, n), 0)
        jj = jax.lax.broadcasted_iota(jnp.int32, (n, n), 1)
        eye = jnp.where(ii == jj, 1.0, 0.0)
        areas_col = jnp.sum(inter * eye, axis=1, keepdims=True)
        areas_row = jnp.transpose(areas_col, (1, 0))
        s2_row = jnp.transpose(s2_col, (1, 0))

        ov = (5.0 * inter > areas_col) | (5.0 * inter > areas_row)
        validp = (s2_col > 0.5) & (s2_row > 0.5)
        cond = ov & (ii < jj) & validp
        smaller_is_i = areas_col < areas_row
        condf = cond.astype(jnp.float32)
        smallf = smaller_is_i.astype(jnp.float32)
        rem_i = jnp.max(condf * smallf, axis=1, keepdims=True)
        rem_j_row = jnp.max(condf * (1.0 - smallf), axis=0, keepdims=True)
        rem_j = jnp.transpose(rem_j_row, (1, 0))
        removed = (rem_i > 0.5) | (rem_j > 0.5)

        big = (200.0 * areas_col) >= float(img_h * img_w)
        s4 = (s2_col > 0.5) & (~removed) & big

        ii_col = jax.lax.broadcasted_iota(jnp.int32, (n, 1), 0)
        cand = jnp.where(s4, ii_col, n)
        first = jnp.min(cand)
        first = jnp.where(first >= n, 0, first)
        copy = pltpu.make_async_copy(
            masks_hbm_ref.at[first], out_ref.at[0], sem
        )
        copy.start()
        copy.wait()


def kernel(masks, bboxes, img_h, img_w):
    n, hh, ww = masks.shape
    hb = 8
    k_steps = hh // hb

    import functools

    out = pl.pallas_call(
        functools.partial(
            _nms_kernel, n=n, k_steps=k_steps, hb=hb, img_h=hh, img_w=ww
        ),
        grid=(k_steps,),
        in_specs=[
            pl.BlockSpec((n, hb, ww), lambda k: (0, k, 0)),
            pl.BlockSpec((n, 4), lambda k: (0, 0)),
            pl.BlockSpec(memory_space=pltpu.MemorySpace.HBM),
        ],
        out_specs=pl.BlockSpec((1, hh, ww), lambda k: (0, 0, 0)),
        out_shape=jax.ShapeDtypeStruct((1, hh, ww), jnp.float32),
        scratch_shapes=[
            pltpu.VMEM((n, n), jnp.float32),
            pltpu.VMEM((n, 1), jnp.float32),
            pltpu.SemaphoreType.DMA,
        ],
    )(masks, bboxes, masks)
    return out.reshape(hh, ww)


# speedup vs baseline: 1.0116x; 1.0116x over previous
import jax
import jax.numpy as jnp
from jax.experimental import pallas as pl
from jax.experimental.pallas import tpu as pltpu


def _nms_kernel(masks_ref, bb_ref, masks_hbm_ref, out_ref, acc_ref, sem, *, n, k_steps, hb, img_h, img_w):
    k = pl.program_id(0)

    @pl.when(k == 0)
    def _init():
        acc_ref[...] = jnp.zeros_like(acc_ref)

    blk = masks_ref[...]
    a = blk.reshape(blk.shape[0], blk.shape[1] * blk.shape[2]).astype(jnp.bfloat16)
    part = jax.lax.dot_general(
        a, a, (((1,), (1,)), ((), ())), preferred_element_type=jnp.float32
    )
    acc_ref[...] += part

    @pl.when(k == k_steps - 1)
    def _epilogue():
        inter = acc_ref[...]
        b = bb_ref[...]
        w = b[:, 2:3] - b[:, 0:1]
        h = b[:, 3:4] - b[:, 1:2]
        keep1 = (h * (1.0 / img_h)) <= 0.95
        keep2 = (w * (1.0 / img_w)) <= 0.95

        ii = jax.lax.broadcasted_iota(jnp.int32, (n, n), 0)
        jj = jax.lax.broadcasted_iota(jnp.int32, (n, n), 1)

        tri = jnp.where(jj <= ii, 1.0, 0.0).astype(jnp.bfloat16)
        keep1f = keep1.astype(jnp.bfloat16)
        rank1 = (
            jax.lax.dot_general(
                tri, keep1f, (((1,), (0,)), ((), ())),
                preferred_element_type=jnp.float32,
            )
            - 1.0
        )
        oh = jnp.where(jj.astype(jnp.float32) == rank1, 1.0, 0.0).astype(
            jnp.bfloat16
        )
        keep2f = keep2.astype(jnp.bfloat16)
        gathered = jax.lax.dot_general(
            oh, keep2f, (((1,), (0,)), ((), ())),
            preferred_element_type=jnp.float32,
        )
        s2_col = jnp.where(keep1, gathered, 0.0)

        eye = jnp.where(ii == jj, 1.0, 0.0)
        areas_col = jnp.sum(inter * eye, axis=1, keepdims=True)
        areas_row = jnp.transpose(areas_col, (1, 0))
        s2_row = jnp.transpose(s2_col, (1, 0))

        ov = (5.0 * inter > areas_col) | (5.0 * inter > areas_row)
        validp = (s2_col > 0.5) & (s2_row > 0.5)
        cond = ov & (ii < jj) & validp
        smaller_is_i = areas_col < areas_row
        condf = cond.astype(jnp.float32)
        smallf = smaller_is_i.astype(jnp.float32)
        rem_i = jnp.max(condf * smallf, axis=1, keepdims=True)
        rem_j_row = jnp.max(condf * (1.0 - smallf), axis=0, keepdims=True)
        rem_j = jnp.transpose(rem_j_row, (1, 0))
        removed = (rem_i > 0.5) | (rem_j > 0.5)

        big = (200.0 * areas_col) >= float(img_h * img_w)
        s4 = (s2_col > 0.5) & (~removed) & big

        ii_col = jax.lax.broadcasted_iota(jnp.int32, (n, 1), 0)
        cand = jnp.where(s4, ii_col, n)
        first = jnp.min(cand)
        first = jnp.where(first >= n, 0, first)
        copy = pltpu.make_async_copy(
            masks_hbm_ref.at[first], out_ref.at[0], sem
        )
        copy.start()
        copy.wait()


def kernel(masks, bboxes, img_h, img_w):
    n, hh, ww = masks.shape
    hb = 8
    k_steps = hh // hb

    import functools

    out = pl.pallas_call(
        functools.partial(
            _nms_kernel, n=n, k_steps=k_steps, hb=hb, img_h=hh, img_w=ww
        ),
        grid=(k_steps,),
        in_specs=[
            pl.BlockSpec((n, hb, ww), lambda k: (0, k, 0)),
            pl.BlockSpec((n, 4), lambda k: (0, 0)),
            pl.BlockSpec(memory_space=pltpu.MemorySpace.HBM),
        ],
        out_specs=pl.BlockSpec((1, hh, ww), lambda k: (0, 0, 0)),
        out_shape=jax.ShapeDtypeStruct((1, hh, ww), jnp.float32),
        scratch_shapes=[
            pltpu.VMEM((n, n), jnp.float32),
            pltpu.SemaphoreType.DMA,
        ],
    )(masks, bboxes, masks)
    return out.reshape(hh, ww)


# speedup vs baseline: 1.0144x; 1.0027x over previous
import functools

import jax
import jax.numpy as jnp
from jax.experimental import pallas as pl
from jax.experimental.pallas import tpu as pltpu


def _nms_kernel(masks_ref, bb_ref, masks_hbm_ref, out_ref, acc_ref, sem, *, n, k_steps, hb, img_h, img_w):
    k = pl.program_id(0)

    @pl.when(k == 0)
    def _init():
        acc_ref[...] = jnp.zeros_like(acc_ref)

    blk = masks_ref[...]
    a = blk.reshape(blk.shape[0], blk.shape[1] * blk.shape[2]).astype(jnp.bfloat16)
    part = jax.lax.dot_general(
        a, a, (((1,), (1,)), ((), ())), preferred_element_type=jnp.float32
    )
    acc_ref[...] += part

    @pl.when(k == k_steps - 1)
    def _epilogue():
        inter = acc_ref[...]
        b = bb_ref[...]
        w = b[:, 2:3] - b[:, 0:1]
        h = b[:, 3:4] - b[:, 1:2]
        keep1 = (h * (1.0 / img_h)) <= 0.95
        keep2 = (w * (1.0 / img_w)) <= 0.95

        ii = jax.lax.broadcasted_iota(jnp.int32, (n, n), 0)
        jj = jax.lax.broadcasted_iota(jnp.int32, (n, n), 1)

        tri = jnp.where(jj <= ii, 1.0, 0.0).astype(jnp.bfloat16)
        keep1f = keep1.astype(jnp.bfloat16)
        rank1 = (
            jax.lax.dot_general(
                tri, keep1f, (((1,), (0,)), ((), ())),
                preferred_element_type=jnp.float32,
            )
            - 1.0
        )
        oh = jnp.where(jj.astype(jnp.float32) == rank1, 1.0, 0.0).astype(
            jnp.bfloat16
        )
        keep2f = keep2.astype(jnp.bfloat16)
        gathered = jax.lax.dot_general(
            oh, keep2f, (((1,), (0,)), ((), ())),
            preferred_element_type=jnp.float32,
        )
        s2_col = jnp.where(keep1, gathered, 0.0)

        eye = jnp.where(ii == jj, 1.0, 0.0)
        areas_col = jnp.sum(inter * eye, axis=1, keepdims=True)
        areas_row = jnp.transpose(areas_col, (1, 0))
        s2_row = jnp.transpose(s2_col, (1, 0))

        ov = (5.0 * inter > areas_col) | (5.0 * inter > areas_row)
        validp = (s2_col > 0.5) & (s2_row > 0.5)
        cond = ov & (ii < jj) & validp
        smaller_is_i = areas_col < areas_row
        condf = cond.astype(jnp.float32)
        smallf = smaller_is_i.astype(jnp.float32)
        rem_i = jnp.max(condf * smallf, axis=1, keepdims=True)
        rem_j_row = jnp.max(condf * (1.0 - smallf), axis=0, keepdims=True)
        rem_j = jnp.transpose(rem_j_row, (1, 0))
        removed = (rem_i > 0.5) | (rem_j > 0.5)

        big = (200.0 * areas_col) >= float(img_h * img_w)
        s4 = (s2_col > 0.5) & (~removed) & big

        ii_col = jax.lax.broadcasted_iota(jnp.int32, (n, 1), 0)
        cand = jnp.where(s4, ii_col, n)
        first = jnp.min(cand)
        first = jnp.where(first >= n, 0, first)
        copy = pltpu.make_async_copy(
            masks_hbm_ref.at[first], out_ref.at[0], sem
        )
        copy.start()
        copy.wait()


def kernel(masks, bboxes, img_h, img_w):
    n, hh, ww = masks.shape
    hb = 8
    k_steps = hh // hb

    out = pl.pallas_call(
        functools.partial(
            _nms_kernel, n=n, k_steps=k_steps, hb=hb, img_h=hh, img_w=ww
        ),
        grid=(k_steps,),
        in_specs=[
            pl.BlockSpec((n, hb, ww), lambda k: (0, k, 0)),
            pl.BlockSpec((n, 4), lambda k: (0, 0)),
            pl.BlockSpec(memory_space=pltpu.MemorySpace.HBM),
        ],
        out_specs=pl.BlockSpec((1, hh, ww), lambda k: (0, 0, 0)),
        out_shape=jax.ShapeDtypeStruct((1, hh, ww), jnp.float32),
        scratch_shapes=[
            pltpu.VMEM((n, n), jnp.float32),
            pltpu.SemaphoreType.DMA,
        ],
    )(masks, bboxes, masks)
    return out.reshape(hh, ww)
